# baseline (speedup 1.0000x reference)
import functools

import jax
import jax.numpy as jnp
from jax import lax
from jax.experimental import pallas as pl
from jax.experimental.pallas import tpu as pltpu
from jax.experimental.pallas import tpu_sc as plsc

N_LEVELS = 8
D_MODEL = 2048

try:
    _info = plsc.get_sparse_core_info()
    _NC, _NS = _info.num_cores, _info.num_subcores
except Exception:
    _NC, _NS = 2, 16
_NW = _NC * _NS


@functools.lru_cache(maxsize=None)
def _build(B: int, D: int):
    b_per_w = B // _NW
    CH = 32
    n_chunks = b_per_w // CH
    mesh = plsc.VectorSubcoreMesh(core_axis_name="c", subcore_axis_name="s")

    @functools.partial(
        pl.kernel,
        mesh=mesh,
        out_type=jax.ShapeDtypeStruct((B, D), jnp.float32),
        scratch_types=[
            pltpu.VMEM((b_per_w,), jnp.int32),
            pltpu.VMEM((CH, D), jnp.float32),
            pltpu.SemaphoreType.DMA,
        ],
    )
    def gather_kernel(idx_hbm, table_hbm, out_hbm, idx_v, buf, sem):
        wid = lax.axis_index("s") * _NC + lax.axis_index("c")
        base = wid * b_per_w
        pltpu.sync_copy(idx_hbm.at[pl.ds(base, b_per_w)], idx_v)
        for j in range(n_chunks):
            pltpu.async_copy(
                table_hbm.at[idx_v.at[pl.ds(j * CH, CH)]], buf, sem
            ).wait()
            pltpu.sync_copy(buf, out_hbm.at[pl.ds(base + j * CH, CH)])

    return gather_kernel


def kernel(level_idx, prefix_emb):
    B = level_idx.shape[0]
    D = prefix_emb.shape[1]
    out = _build(B, D)(level_idx, prefix_emb)
    return out.reshape(B, 1, D)


# speedup vs baseline: 1.0025x; 1.0025x over previous
import functools

import jax
import jax.numpy as jnp
from jax import lax
from jax.experimental import pallas as pl
from jax.experimental.pallas import tpu as pltpu
from jax.experimental.pallas import tpu_sc as plsc

N_LEVELS = 8
D_MODEL = 2048

try:
    _info = plsc.get_sparse_core_info()
    _NC, _NS = _info.num_cores, _info.num_subcores
except Exception:
    _NC, _NS = 2, 16
_NW = _NC * _NS


@functools.lru_cache(maxsize=None)
def _build(B: int, D: int):
    b_per_w = B // _NW
    CH = 16
    NBUF = 3
    n_chunks = b_per_w // CH
    mesh = plsc.VectorSubcoreMesh(core_axis_name="c", subcore_axis_name="s")

    @functools.partial(
        pl.kernel,
        mesh=mesh,
        out_type=jax.ShapeDtypeStruct((B, D), jnp.float32),
        scratch_types=[
            pltpu.VMEM((b_per_w,), jnp.int32),
            [pltpu.VMEM((CH, D), jnp.float32) for _ in range(NBUF)],
            [pltpu.SemaphoreType.DMA for _ in range(NBUF)],
            [pltpu.SemaphoreType.DMA for _ in range(NBUF)],
        ],
    )
    def gather_kernel(idx_hbm, table_hbm, out_hbm, idx_v, bufs, gsems, wsems):
        wid = lax.axis_index("s") * _NC + lax.axis_index("c")
        base = wid * b_per_w
        pltpu.sync_copy(idx_hbm.at[pl.ds(base, b_per_w)], idx_v)

        def start_gather(j):
            return pltpu.async_copy(
                table_hbm.at[idx_v.at[pl.ds(j * CH, CH)]],
                bufs[j % NBUF], gsems[j % NBUF])

        gh = [None] * n_chunks
        wh = [None] * n_chunks
        gh[0] = start_gather(0)
        for j in range(n_chunks):
            if j + 1 < n_chunks:
                if j + 1 >= NBUF:
                    wh[j + 1 - NBUF].wait()
                gh[j + 1] = start_gather(j + 1)
            gh[j].wait()
            wh[j] = pltpu.async_copy(
                bufs[j % NBUF], out_hbm.at[pl.ds(base + j * CH, CH)],
                wsems[j % NBUF])
        for j in range(max(0, n_chunks - NBUF), n_chunks):
            wh[j].wait()

    return gather_kernel


def kernel(level_idx, prefix_emb):
    B = level_idx.shape[0]
    D = prefix_emb.shape[1]
    out = _build(B, D)(level_idx, prefix_emb)
    return out.reshape(B, 1, D)


# speedup vs baseline: 1.2982x; 1.2949x over previous
import functools

import jax
import jax.numpy as jnp
from jax import lax
from jax.experimental import pallas as pl
from jax.experimental.pallas import tpu as pltpu
from jax.experimental.pallas import tpu_sc as plsc

N_LEVELS = 8
D_MODEL = 2048

try:
    _info = plsc.get_sparse_core_info()
    _NC, _NS = _info.num_cores, _info.num_subcores
except Exception:
    _NC, _NS = 2, 16
_NW = _NC * _NS


@functools.lru_cache(maxsize=None)
def _build(B: int, D: int):
    b_per_w = B // _NW
    CH = 16
    NBUF = 3
    n_chunks = b_per_w // CH
    mesh = plsc.VectorSubcoreMesh(core_axis_name="c", subcore_axis_name="s")

    @functools.partial(
        pl.kernel,
        mesh=mesh,
        out_type=jax.ShapeDtypeStruct((B, 1, D), jnp.float32),
        scratch_types=[
            pltpu.VMEM((b_per_w,), jnp.int32),
            [pltpu.VMEM((CH, D), jnp.float32) for _ in range(NBUF)],
            [pltpu.SemaphoreType.DMA for _ in range(NBUF)],
            [pltpu.SemaphoreType.DMA for _ in range(NBUF)],
        ],
    )
    def gather_kernel(idx_hbm, table_hbm, out_hbm, idx_v, bufs, gsems, wsems):
        wid = lax.axis_index("s") * _NC + lax.axis_index("c")
        base = wid * b_per_w
        pltpu.sync_copy(idx_hbm.at[pl.ds(base, b_per_w)], idx_v)

        def start_gather(j):
            return pltpu.async_copy(
                table_hbm.at[idx_v.at[pl.ds(j * CH, CH)]],
                bufs[j % NBUF], gsems[j % NBUF])

        gh = [None] * n_chunks
        wh = [None] * n_chunks
        gh[0] = start_gather(0)
        for j in range(n_chunks):
            if j + 1 < n_chunks:
                if j + 1 >= NBUF:
                    wh[j + 1 - NBUF].wait()
                gh[j + 1] = start_gather(j + 1)
            gh[j].wait()
            wh[j] = pltpu.async_copy(
                bufs[j % NBUF], out_hbm.at[pl.ds(base + j * CH, CH), 0],
                wsems[j % NBUF])
        for j in range(max(0, n_chunks - NBUF), n_chunks):
            wh[j].wait()

    return gather_kernel


def kernel(level_idx, prefix_emb):
    B = level_idx.shape[0]
    D = prefix_emb.shape[1]
    return _build(B, D)(level_idx, prefix_emb)


# speedup vs baseline: 3.2636x; 2.5140x over previous
import functools

import jax
import jax.numpy as jnp
from jax import lax
from jax.experimental import pallas as pl
from jax.experimental.pallas import tpu as pltpu
from jax.experimental.pallas import tpu_sc as plsc

N_LEVELS = 8
D_MODEL = 2048

try:
    _info = plsc.get_sparse_core_info()
    _NC, _NS = _info.num_cores, _info.num_subcores
except Exception:
    _NC, _NS = 2, 16
_NW = _NC * _NS


@functools.lru_cache(maxsize=None)
def _build(B: int, D: int, V: int):
    b_per_w = B // _NW
    CH = 16
    NBUF = 3
    n_chunks = b_per_w // CH
    mesh = plsc.VectorSubcoreMesh(core_axis_name="c", subcore_axis_name="s")

    @functools.partial(
        pl.kernel,
        mesh=mesh,
        out_type=jax.ShapeDtypeStruct((B, 1, D), jnp.float32),
        scratch_types=[
            pltpu.VMEM((b_per_w,), jnp.int32),
            [pltpu.VMEM((CH, D), jnp.float32) for _ in range(NBUF)],
            [pltpu.SemaphoreType.DMA for _ in range(NBUF)],
            [pltpu.SemaphoreType.DMA for _ in range(NBUF)],
        ],
    )
    def gather_kernel(idx_hbm, rep_hbm, out_hbm, idx_v, bufs, gsems, wsems):
        wid = lax.axis_index("s") * _NC + lax.axis_index("c")
        base = wid * b_per_w
        pltpu.sync_copy(idx_hbm.at[pl.ds(base, b_per_w)], idx_v)
        row_off = wid * V

        def start_gather(j):
            iv = idx_v[pl.ds(j * CH, CH)] + row_off
            return pltpu.async_copy(
                rep_hbm.at[iv], bufs[j % NBUF], gsems[j % NBUF])

        gh = [None] * n_chunks
        wh = [None] * n_chunks
        gh[0] = start_gather(0)
        for j in range(n_chunks):
            if j + 1 < n_chunks:
                if j + 1 >= NBUF:
                    wh[j + 1 - NBUF].wait()
                gh[j + 1] = start_gather(j + 1)
            gh[j].wait()
            wh[j] = pltpu.async_copy(
                bufs[j % NBUF], out_hbm.at[pl.ds(base + j * CH, CH), 0],
                wsems[j % NBUF])
        for j in range(max(0, n_chunks - NBUF), n_chunks):
            wh[j].wait()

    return gather_kernel


def kernel(level_idx, prefix_emb):
    B = level_idx.shape[0]
    V, D = prefix_emb.shape
    rep = jnp.broadcast_to(prefix_emb[None], (_NW, V, D)).reshape(_NW * V, D)
    return _build(B, D, V)(level_idx, rep)
